# Initial kernel scaffold; baseline (speedup 1.0000x reference)
import jax
import jax.numpy as jnp
from jax.experimental import pallas as pl


def kernel(yHat, y_fine, fine2coarse):
    raise NotImplementedError("write your pallas kernel here")


# speedup vs baseline: 19.0340x; 19.0340x over previous
import functools

import jax
import jax.numpy as jnp
from jax.experimental import pallas as pl
from jax.experimental.pallas import tpu as pltpu

_B = 16384
_C = 100
_BS = 1024


def _loss_body(yhat_ref, yf_ref, f2c_ref, out_ref, *, scale):
    x = yhat_ref[...]
    yf = yf_ref[...]
    f2c = f2c_ref[...]

    bs, c = x.shape
    col = jax.lax.broadcasted_iota(jnp.int32, (bs, c), 1)
    onehot = col == yf
    y_coarse = jnp.sum(jnp.where(onehot, f2c, 0), axis=1, keepdims=True)
    mask = f2c == y_coarse

    m = jnp.max(x, axis=1, keepdims=True)
    e = jnp.exp(x - m)
    z = jnp.sum(e, axis=1, keepdims=True)
    e_in = jnp.sum(jnp.where(mask, e, 0.0), axis=1, keepdims=True)
    yg = e_in / z
    yg_ = 1.0 - yg + 1e-7
    rscale = 1.0 / (z * yg_)
    px = e * rscale
    px_log = jnp.log(jnp.maximum(px, 1e-10))
    contrib = jnp.where(mask, 0.0, px * px_log)
    partial = jnp.sum(contrib) * scale

    @pl.when(pl.program_id(0) == 0)
    def _():
        out_ref[0, 0] = 0.0

    out_ref[0, 0] += partial


@jax.jit
def kernel(yHat, y_fine, fine2coarse):
    b, c = yHat.shape
    scale = 1.0 / (float(b) * float(c))
    grid = b // _BS
    out = pl.pallas_call(
        functools.partial(_loss_body, scale=scale),
        grid=(grid,),
        in_specs=[
            pl.BlockSpec((_BS, c), lambda i: (i, 0)),
            pl.BlockSpec((_BS, 1), lambda i: (i, 0)),
            pl.BlockSpec((1, c), lambda i: (0, 0)),
        ],
        out_specs=pl.BlockSpec(memory_space=pltpu.SMEM),
        out_shape=jax.ShapeDtypeStruct((1, 1), jnp.float32),
    )(yHat, y_fine.reshape(b, 1), fine2coarse.reshape(1, c))
    return out[0, 0]
